# baseline (speedup 1.0000x reference)
import functools

import jax
import jax.numpy as jnp
from jax import lax
from jax.experimental import pallas as pl
from jax.experimental.pallas import tpu as pltpu
from jax.experimental.pallas import tpu_sc as plsc

V = 100000
E = 1024
H = 1024
B = 128

_SUB = 512
_NFULL = V // _SUB
_TAIL = V - _NFULL * _SUB
_SPB = 4
_BV = _SPB * _SUB
_NV = V // _BV + 1
_NBUF = 8
_CHUNK = 512

_NC = 2
_GW = 16
_RPW = B // _GW

_nt_dims = (((1,), (1,)), ((), ()))


def _sc_gather(ids, emb):
    mesh = plsc.VectorSubcoreMesh(core_axis_name="c", subcore_axis_name="s")

    @functools.partial(
        pl.kernel,
        mesh=mesh,
        out_type=jax.ShapeDtypeStruct((B, E), jnp.float32),
        scratch_types=[
            pltpu.VMEM((_RPW,), jnp.int32),
            pltpu.VMEM((_RPW, E), jnp.float32),
            pltpu.SemaphoreType.DMA,
        ],
    )
    def gather_kernel(ids_hbm, emb_hbm, x_hbm, idx_v, rows_v, sem):
        wid = lax.axis_index("s") * _NC + lax.axis_index("c")

        @pl.when(wid < _GW)
        def _():
            base = wid * _RPW
            pltpu.sync_copy(ids_hbm.at[pl.ds(base, _RPW)], idx_v)
            pltpu.async_copy(emb_hbm.at[idx_v], rows_v, sem).wait()
            pltpu.sync_copy(rows_v, x_hbm.at[pl.ds(base, _RPW)])

    return gather_kernel(ids, emb)


def _lstm_body(x_ref, h_ref, c_ref, wih_ref, whh_ref, bih_ref, bhh_ref,
               hout_ref, cout_ref):
    gates = (
        lax.dot_general(x_ref[...], wih_ref[...], _nt_dims,
                        preferred_element_type=jnp.float32)
        + lax.dot_general(h_ref[...], whh_ref[...], _nt_dims,
                          preferred_element_type=jnp.float32)
        + bih_ref[...] + bhh_ref[...]
    )
    i_g = jax.nn.sigmoid(gates[:, 0:H])
    f_g = jax.nn.sigmoid(gates[:, H:2 * H])
    g_g = jnp.tanh(gates[:, 2 * H:3 * H])
    o_g = jax.nn.sigmoid(gates[:, 3 * H:4 * H])
    c_new = f_g * c_ref[...] + i_g * g_g
    cout_ref[...] = c_new
    hout_ref[...] = o_g * jnp.tanh(c_new)


def _lstm(x, h, c, W_ih, W_hh, b_ih2, b_hh2):
    return pl.pallas_call(
        _lstm_body,
        out_shape=[
            jax.ShapeDtypeStruct((B, H), jnp.float32),
            jax.ShapeDtypeStruct((B, H), jnp.float32),
        ],
    )(x, h, c, W_ih, W_hh, b_ih2, b_hh2)


def _fc_body(h_ref, fcb_ref, fcw_hbm, pred_hbm, bufs, otiles, ttail,
             isems, osems, tsem):

    def fire_rows(g, rows):
        slot = lax.rem(g, _NBUF)
        for off in range(0, rows, _CHUNK):
            n = min(_CHUNK, rows - off)
            pltpu.make_async_copy(
                fcw_hbm.at[pl.ds(g * _SUB + off, n)],
                bufs.at[slot, pl.ds(off, n)],
                isems.at[slot]).start()

    def fire_sub(g):
        @pl.when(g < _NFULL)
        def _():
            fire_rows(g, _SUB)

        @pl.when(g == _NFULL)
        def _():
            fire_rows(g, _TAIL)

    def wait_sub(g, rows):
        slot = lax.rem(g, _NBUF)
        pltpu.make_async_copy(
            fcw_hbm.at[pl.ds(g * _SUB, rows)],
            bufs.at[slot, pl.ds(0, rows)],
            isems.at[slot]).wait()

    def out_desc(g, rows):
        oslot = lax.rem(g, _NBUF)
        return pltpu.make_async_copy(
            otiles.at[oslot, slice(None), pl.ds(0, rows)],
            pred_hbm.at[slice(None), pl.ds(g * _SUB, rows)],
            osems.at[oslot])

    for k in range(_NBUF - 1):
        fire_sub(jnp.int32(k))

    def step(g, carry):
        fire_sub(g + _NBUF - 1)
        slot = lax.rem(g, _NBUF)
        wait_sub(g, _SUB)
        tile = (
            lax.dot_general(h_ref[...], bufs[slot], _nt_dims,
                            preferred_element_type=jnp.float32)
            + fcb_ref[g]
        )

        @pl.when(g >= _NBUF)
        def _():
            out_desc(g - _NBUF, _SUB).wait()

        otiles[slot] = tile
        out_desc(g, _SUB).start()
        return carry

    lax.fori_loop(0, _NFULL, step, 0, unroll=4)

    wait_sub(jnp.int32(_NFULL), _TAIL)
    ttile = (
        lax.dot_general(h_ref[...], bufs[_NFULL % _NBUF, 0:_TAIL],
                        _nt_dims, preferred_element_type=jnp.float32)
        + fcb_ref[_NFULL, :, 0:_TAIL]
    )
    ttail[...] = ttile
    tail_copy = pltpu.make_async_copy(
        ttail, pred_hbm.at[slice(None), pl.ds(_NFULL * _SUB, _TAIL)], tsem)
    tail_copy.start()

    for g in range(_NFULL - _NBUF, _NFULL):
        out_desc(jnp.int32(g), _SUB).wait()
    tail_copy.wait()


def _fc(h_new, fc_W, fcb3):
    return pl.pallas_call(
        _fc_body,
        in_specs=[
            pl.BlockSpec(memory_space=pltpu.MemorySpace.VMEM),
            pl.BlockSpec(memory_space=pltpu.MemorySpace.VMEM),
            pl.BlockSpec(memory_space=pltpu.MemorySpace.HBM),
        ],
        out_specs=pl.BlockSpec(memory_space=pltpu.MemorySpace.HBM),
        out_shape=jax.ShapeDtypeStruct((B, V), jnp.float32),
        scratch_shapes=[
            pltpu.VMEM((_NBUF, _SUB, H), jnp.float32),
            pltpu.VMEM((_NBUF, B, _SUB), jnp.float32),
            pltpu.VMEM((B, _TAIL), jnp.float32),
            pltpu.SemaphoreType.DMA((_NBUF,)),
            pltpu.SemaphoreType.DMA((_NBUF,)),
            pltpu.SemaphoreType.DMA,
        ],
        compiler_params=pltpu.CompilerParams(
            vmem_limit_bytes=60 * 1024 * 1024),
    )(h_new, fcb3, fc_W)


def kernel(input, h0, c0, emb, W_ih, W_hh, b_ih, b_hh, fc_W, fc_b):
    ids = input.astype(jnp.int32)
    x = _sc_gather(ids, emb)
    h_new, c_new = _lstm(x, h0[0], c0[0], W_ih, W_hh,
                         b_ih.reshape(1, 4 * H), b_hh.reshape(1, 4 * H))
    fcb3 = jnp.pad(fc_b, (0, (_NFULL + 1) * _SUB - V)).reshape(
        _NFULL + 1, 1, _SUB)
    pred = _fc(h_new, fc_W, fcb3)
    return (pred, h_new[None, :, :], c_new[None, :, :])


# speedup vs baseline: 1.2622x; 1.2622x over previous
import functools

import jax
import jax.numpy as jnp
from jax import lax
from jax.experimental import pallas as pl
from jax.experimental.pallas import tpu as pltpu
from jax.experimental.pallas import tpu_sc as plsc

V = 100000
E = 1024
H = 1024
B = 128

_SUB = 2048
_NFULL = V // _SUB
_TAIL = V - _NFULL * _SUB
_SPB = 4
_BV = _SPB * _SUB
_NV = V // _BV + 1
_NBUF = 4
_CHUNK = 512

_NC = 2
_GW = 16
_RPW = B // _GW

_nt_dims = (((1,), (1,)), ((), ()))


def _sc_gather(ids, emb):
    mesh = plsc.VectorSubcoreMesh(core_axis_name="c", subcore_axis_name="s")

    @functools.partial(
        pl.kernel,
        mesh=mesh,
        out_type=jax.ShapeDtypeStruct((B, E), jnp.float32),
        scratch_types=[
            pltpu.VMEM((_RPW,), jnp.int32),
            pltpu.VMEM((_RPW, E), jnp.float32),
            pltpu.SemaphoreType.DMA,
        ],
    )
    def gather_kernel(ids_hbm, emb_hbm, x_hbm, idx_v, rows_v, sem):
        wid = lax.axis_index("s") * _NC + lax.axis_index("c")

        @pl.when(wid < _GW)
        def _():
            base = wid * _RPW
            pltpu.sync_copy(ids_hbm.at[pl.ds(base, _RPW)], idx_v)
            pltpu.async_copy(emb_hbm.at[idx_v], rows_v, sem).wait()
            pltpu.sync_copy(rows_v, x_hbm.at[pl.ds(base, _RPW)])

    return gather_kernel(ids, emb)


def _lstm_body(x_ref, h_ref, c_ref, wih_ref, whh_ref, bih_ref, bhh_ref,
               hout_ref, cout_ref):
    gates = (
        lax.dot_general(x_ref[...], wih_ref[...], _nt_dims,
                        preferred_element_type=jnp.float32)
        + lax.dot_general(h_ref[...], whh_ref[...], _nt_dims,
                          preferred_element_type=jnp.float32)
        + bih_ref[...] + bhh_ref[...]
    )
    i_g = jax.nn.sigmoid(gates[:, 0:H])
    f_g = jax.nn.sigmoid(gates[:, H:2 * H])
    g_g = jnp.tanh(gates[:, 2 * H:3 * H])
    o_g = jax.nn.sigmoid(gates[:, 3 * H:4 * H])
    c_new = f_g * c_ref[...] + i_g * g_g
    cout_ref[...] = c_new
    hout_ref[...] = o_g * jnp.tanh(c_new)


def _lstm(x, h, c, W_ih, W_hh, b_ih2, b_hh2):
    return pl.pallas_call(
        _lstm_body,
        out_shape=[
            jax.ShapeDtypeStruct((B, H), jnp.float32),
            jax.ShapeDtypeStruct((B, H), jnp.float32),
        ],
    )(x, h, c, W_ih, W_hh, b_ih2, b_hh2)


def _fc_body(h_ref, fcb_ref, fcw_hbm, predt_hbm, bufs, otiles, isems, osems):

    def fire_rows(g, rows):
        slot = lax.rem(g, _NBUF)
        for off in range(0, rows, _CHUNK):
            n = min(_CHUNK, rows - off)
            pltpu.make_async_copy(
                fcw_hbm.at[pl.ds(g * _SUB + off, n)],
                bufs.at[slot, pl.ds(off, n)],
                isems.at[slot]).start()

    def fire_sub(g):
        @pl.when(g < _NFULL)
        def _():
            fire_rows(g, _SUB)

        @pl.when(g == _NFULL)
        def _():
            fire_rows(g, _TAIL)

    def wait_sub(g, rows):
        slot = lax.rem(g, _NBUF)
        pltpu.make_async_copy(
            fcw_hbm.at[pl.ds(g * _SUB, rows)],
            bufs.at[slot, pl.ds(0, rows)],
            isems.at[slot]).wait()

    def out_desc(g, rows):
        oslot = lax.rem(g, _NBUF)
        return pltpu.make_async_copy(
            otiles.at[oslot, pl.ds(0, rows)],
            predt_hbm.at[pl.ds(g * _SUB, rows)],
            osems.at[oslot])

    for k in range(_NBUF - 1):
        fire_sub(jnp.int32(k))

    def step(g, carry):
        fire_sub(g + _NBUF - 1)
        slot = lax.rem(g, _NBUF)
        wait_sub(g, _SUB)
        tile = (
            lax.dot_general(bufs[slot], h_ref[...], _nt_dims,
                            preferred_element_type=jnp.float32)
            + fcb_ref[g].reshape(_SUB, 1)
        )

        @pl.when(g >= _NBUF)
        def _():
            out_desc(g - _NBUF, _SUB).wait()

        otiles[slot] = tile
        out_desc(g, _SUB).start()
        return carry

    lax.fori_loop(0, _NFULL, step, 0, unroll=4)

    wait_sub(jnp.int32(_NFULL), _TAIL)
    ttile = (
        lax.dot_general(bufs[_NFULL % _NBUF, 0:_TAIL], h_ref[...],
                        _nt_dims, preferred_element_type=jnp.float32)
        + fcb_ref[_NFULL, 0:_TAIL].reshape(_TAIL, 1)
    )
    tslot = _NFULL % _NBUF
    out_desc(jnp.int32(_NFULL - _NBUF), _SUB).wait()
    otiles[tslot, 0:_TAIL] = ttile
    out_desc(jnp.int32(_NFULL), _TAIL).start()

    for g in range(_NFULL - _NBUF + 1, _NFULL):
        out_desc(jnp.int32(g), _SUB).wait()
    out_desc(jnp.int32(_NFULL), _TAIL).wait()


def _fc(h_new, fc_W, fcb3):
    return pl.pallas_call(
        _fc_body,
        in_specs=[
            pl.BlockSpec(memory_space=pltpu.MemorySpace.VMEM),
            pl.BlockSpec(memory_space=pltpu.MemorySpace.VMEM),
            pl.BlockSpec(memory_space=pltpu.MemorySpace.HBM),
        ],
        out_specs=pl.BlockSpec(memory_space=pltpu.MemorySpace.HBM),
        out_shape=jax.ShapeDtypeStruct((V, B), jnp.float32),
        scratch_shapes=[
            pltpu.VMEM((_NBUF, _SUB, H), jnp.float32),
            pltpu.VMEM((_NBUF, _SUB, B), jnp.float32),
            pltpu.SemaphoreType.DMA((_NBUF,)),
            pltpu.SemaphoreType.DMA((_NBUF,)),
        ],
        compiler_params=pltpu.CompilerParams(
            vmem_limit_bytes=60 * 1024 * 1024),
    )(h_new, fcb3, fc_W)


def kernel(input, h0, c0, emb, W_ih, W_hh, b_ih, b_hh, fc_W, fc_b):
    ids = input.astype(jnp.int32)
    x = _sc_gather(ids, emb)
    h_new, c_new = _lstm(x, h0[0], c0[0], W_ih, W_hh,
                         b_ih.reshape(1, 4 * H), b_hh.reshape(1, 4 * H))
    fcb3 = jnp.pad(fc_b, (0, (_NFULL + 1) * _SUB - V)).reshape(
        _NFULL + 1, _SUB)
    pred_t = _fc(h_new, fc_W, fcb3)
    return (pred_t.T, h_new[None, :, :], c_new[None, :, :])
